# baseline (speedup 1.0000x reference)
import functools

import jax
import jax.numpy as jnp
from jax import lax
from jax.experimental import pallas as pl
from jax.experimental.pallas import tpu as pltpu
from jax.experimental.pallas import tpu_sc as plsc

B, S, D = 4, 4096, 2048
NTOK = B * S
NWORKERS = 32
TOK_PER_W = NTOK // NWORKERS
CHUNK = 16
NCHUNK = TOK_PER_W // CHUNK
NVEC = D // 16
EPS = 1e-5


_GDN = lax.GatherDimensionNumbers(
    offset_dims=(), collapsed_slice_dims=(0,), start_index_map=(0,))


def _lane_shuffle(v, idx):
    return lax.gather(v, idx[:, None], dimension_numbers=_GDN, slice_sizes=(1,),
                      mode=lax.GatherScatterMode.PROMISE_IN_BOUNDS)


def _lane_sum(v, lanes):
    for sh in (8, 4, 2, 1):
        v = v + _lane_shuffle(v, lanes ^ sh)
    return v


def _rsqrt_newton(v):
    bits = lax.bitcast_convert_type(v, jnp.int32)
    y = lax.bitcast_convert_type(jnp.int32(0x5F3759DF) - (bits >> 1), jnp.float32)
    for _ in range(3):
        y = y * (1.5 - 0.5 * v * y * y)
    return y


def _sc_body(h_hbm, idx_hbm, w0_hbm, b0_hbm, w1_hbm, b1_hbm, out_hbm,
             buf, w0_v, b0_v, w1_v, b1_v, idx_v, stats_v):
    ncores = plsc.get_sparse_core_info().num_cores
    wid = lax.axis_index("s") * ncores + lax.axis_index("c")
    tok0 = wid * TOK_PER_W

    pltpu.sync_copy(w0_hbm, w0_v)
    pltpu.sync_copy(b0_hbm, b0_v)
    pltpu.sync_copy(w1_hbm, w1_v)
    pltpu.sync_copy(b1_hbm, b1_v)
    pltpu.sync_copy(idx_hbm.at[pl.ds(tok0, TOK_PER_W)], idx_v)

    lanes = lax.iota(jnp.int32, 16)
    zero16 = jnp.zeros((16,), jnp.float32)

    def chunk_body(c, _):
        base_elem = (tok0 + c * CHUNK) * D
        pltpu.sync_copy(h_hbm.at[pl.ds(base_elem, CHUNK * D)], buf)

        def tok_body(t, carry):
            mu_v, var_v = carry

            def acc_body(j, c2):
                vx, vq = c2
                x = buf[pl.ds(t * D + j * 16, 16)]
                return vx + x, vq + x * x

            vx, vq = lax.fori_loop(0, NVEC, acc_body, (zero16, zero16))
            mu = _lane_sum(vx, lanes) * (1.0 / D)
            var = _lane_sum(vq, lanes) * (1.0 / D) - mu * mu
            m = lanes == t
            return jnp.where(m, mu, mu_v), jnp.where(m, var, var_v)

        mu_v, var_v = lax.fori_loop(0, CHUNK, tok_body, (zero16, zero16))
        rstd_v = _rsqrt_newton(var_v + EPS)
        tv_v = idx_v[pl.ds(c * CHUNK, 16)]
        stats_v[pl.ds(0, 16)] = mu_v
        stats_v[pl.ds(16, 16)] = rstd_v
        stats_v[pl.ds(32, 16)] = tv_v

        def norm_body(t, _):
            mu_s = plsc.load_gather(stats_v, [jnp.broadcast_to(t, (16,))])
            rstd_s = plsc.load_gather(stats_v, [jnp.broadcast_to(t + 16, (16,))])
            tv_s = plsc.load_gather(stats_v, [jnp.broadcast_to(t + 32, (16,))])
            sel1 = tv_s != 0.0

            def col_body(j, _):
                off = t * D + j * 16
                x = buf[pl.ds(off, 16)]
                wj = jnp.where(sel1, w1_v[pl.ds(j * 16, 16)], w0_v[pl.ds(j * 16, 16)])
                bj = jnp.where(sel1, b1_v[pl.ds(j * 16, 16)], b0_v[pl.ds(j * 16, 16)])
                buf[pl.ds(off, 16)] = (x - mu_s) * rstd_s * wj + bj
                return 0

            lax.fori_loop(0, NVEC, col_body, 0)
            return 0

        lax.fori_loop(0, CHUNK, norm_body, 0)

        pltpu.sync_copy(buf, out_hbm.at[pl.ds(base_elem, CHUNK * D)])
        return 0

    lax.fori_loop(0, NCHUNK, chunk_body, 0)


@jax.jit
def kernel(hidden_states, multiway_indices, ln0_w, ln0_b, ln1_w, ln1_b):
    h_flat = hidden_states.reshape(-1)
    idx_flat = multiway_indices.reshape(-1).astype(jnp.float32)

    mesh = plsc.VectorSubcoreMesh(core_axis_name="c", subcore_axis_name="s")
    run = pl.kernel(
        _sc_body,
        out_type=jax.ShapeDtypeStruct((NTOK * D,), jnp.float32),
        mesh=mesh,
        compiler_params=pltpu.CompilerParams(needs_layout_passes=False),
        scratch_types=[
            pltpu.VMEM((CHUNK * D,), jnp.float32),
            pltpu.VMEM((D,), jnp.float32),
            pltpu.VMEM((D,), jnp.float32),
            pltpu.VMEM((D,), jnp.float32),
            pltpu.VMEM((D,), jnp.float32),
            pltpu.VMEM((TOK_PER_W,), jnp.float32),
            pltpu.VMEM((48,), jnp.float32),
        ],
    )
    out = run(h_flat, idx_flat, ln0_w, ln0_b, ln1_w, ln1_b)
    return out.reshape(B, S, D)


# speedup vs baseline: 2.0502x; 2.0502x over previous
import functools

import jax
import jax.numpy as jnp
from jax import lax
from jax.experimental import pallas as pl
from jax.experimental.pallas import tpu as pltpu
from jax.experimental.pallas import tpu_sc as plsc

B, S, D = 4, 4096, 2048
NTOK = B * S
NWORKERS = 32
TOK_PER_W = NTOK // NWORKERS
CHUNK = 16
NCHUNK = TOK_PER_W // CHUNK
NVEC = D // 16
EPS = 1e-5


_GDN = lax.GatherDimensionNumbers(
    offset_dims=(), collapsed_slice_dims=(0,), start_index_map=(0,))


def _lane_shuffle(v, idx):
    return lax.gather(v, idx[:, None], dimension_numbers=_GDN, slice_sizes=(1,),
                      mode=lax.GatherScatterMode.PROMISE_IN_BOUNDS)


def _lane_sum(v, lanes):
    for sh in (8, 4, 2, 1):
        v = v + _lane_shuffle(v, lanes ^ sh)
    return v


def _rsqrt_newton(v):
    bits = lax.bitcast_convert_type(v, jnp.int32)
    y = lax.bitcast_convert_type(jnp.int32(0x5F3759DF) - (bits >> 1), jnp.float32)
    for _ in range(3):
        y = y * (1.5 - 0.5 * v * y * y)
    return y


def _sc_body(h_hbm, idx_hbm, w0_hbm, b0_hbm, w1_hbm, b1_hbm, out_hbm,
             buf, w0_v, b0_v, w1_v, b1_v, idx_v, stats_v):
    ncores = plsc.get_sparse_core_info().num_cores
    wid = lax.axis_index("s") * ncores + lax.axis_index("c")
    tok0 = wid * TOK_PER_W

    pltpu.sync_copy(w0_hbm, w0_v)
    pltpu.sync_copy(b0_hbm, b0_v)
    pltpu.sync_copy(w1_hbm, w1_v)
    pltpu.sync_copy(b1_hbm, b1_v)
    pltpu.sync_copy(idx_hbm.at[pl.ds(tok0, TOK_PER_W)], idx_v)

    lanes = lax.iota(jnp.int32, 16)
    zero16 = jnp.zeros((16,), jnp.float32)

    NSEG = 16
    JSEG = NVEC // NSEG

    def chunk_body(c, _):
        base_elem = (tok0 + c * CHUNK) * D
        pltpu.sync_copy(h_hbm.at[pl.ds(base_elem, CHUNK * D)], buf)

        def tok_body(t, carry):
            mu_v, var_v = carry
            row = t * D
            acc = [zero16, zero16, zero16, zero16]
            for j in range(NVEC):
                x = buf[pl.ds(row + j * 16, 16)]
                k = j & 1
                acc[k] = acc[k] + x
                acc[2 + k] = acc[2 + k] + x * x
            mu = _lane_sum(acc[0] + acc[1], lanes) * (1.0 / D)
            var = _lane_sum(acc[2] + acc[3], lanes) * (1.0 / D) - mu * mu
            m = lanes == t
            return jnp.where(m, mu, mu_v), jnp.where(m, var, var_v)

        mu_v, var_v = lax.fori_loop(0, CHUNK, tok_body, (zero16, zero16))
        rstd_v = _rsqrt_newton(var_v + EPS)
        tv_v = idx_v[pl.ds(c * CHUNK, 16)]
        stats_v[pl.ds(16, 16)] = mu_v
        stats_v[pl.ds(32, 16)] = rstd_v
        stats_v[pl.ds(48, 16)] = tv_v

        def seg_body(s, _):
            jb = s * (JSEG * 16)
            w0r = [w0_v[pl.ds(jb + k * 16, 16)] for k in range(JSEG)]
            w1r = [w1_v[pl.ds(jb + k * 16, 16)] for k in range(JSEG)]
            b0r = [b0_v[pl.ds(jb + k * 16, 16)] for k in range(JSEG)]
            b1r = [b1_v[pl.ds(jb + k * 16, 16)] for k in range(JSEG)]
            for t in range(CHUNK):
                mu_s = plsc.load_gather(stats_v, [jnp.full((16,), 16 + t, jnp.int32)])
                rstd_s = plsc.load_gather(stats_v, [jnp.full((16,), 32 + t, jnp.int32)])
                tv_s = plsc.load_gather(stats_v, [jnp.full((16,), 48 + t, jnp.int32)])
                sel1 = tv_s != 0.0
                off = t * D + jb
                for k in range(JSEG):
                    x = buf[pl.ds(off + k * 16, 16)]
                    wj = jnp.where(sel1, w1r[k], w0r[k])
                    bj = jnp.where(sel1, b1r[k], b0r[k])
                    buf[pl.ds(off + k * 16, 16)] = (x - mu_s) * rstd_s * wj + bj
            return 0

        lax.fori_loop(0, NSEG, seg_body, 0)

        pltpu.sync_copy(buf, out_hbm.at[pl.ds(base_elem, CHUNK * D)])
        return 0

    lax.fori_loop(0, NCHUNK, chunk_body, 0)


@jax.jit
def kernel(hidden_states, multiway_indices, ln0_w, ln0_b, ln1_w, ln1_b):
    h_flat = hidden_states.reshape(-1)
    idx_flat = multiway_indices.reshape(-1).astype(jnp.float32)

    mesh = plsc.VectorSubcoreMesh(core_axis_name="c", subcore_axis_name="s")
    run = pl.kernel(
        _sc_body,
        out_type=jax.ShapeDtypeStruct((NTOK * D,), jnp.float32),
        mesh=mesh,
        compiler_params=pltpu.CompilerParams(needs_layout_passes=False),
        scratch_types=[
            pltpu.VMEM((CHUNK * D,), jnp.float32),
            pltpu.VMEM((D,), jnp.float32),
            pltpu.VMEM((D,), jnp.float32),
            pltpu.VMEM((D,), jnp.float32),
            pltpu.VMEM((D,), jnp.float32),
            pltpu.VMEM((TOK_PER_W,), jnp.float32),
            pltpu.VMEM((64,), jnp.float32),
        ],
    )
    out = run(h_flat, idx_flat, ln0_w, ln0_b, ln1_w, ln1_b)
    return out.reshape(B, S, D)


# speedup vs baseline: 2.6869x; 1.3106x over previous
import functools

import jax
import jax.numpy as jnp
from jax import lax
from jax.experimental import pallas as pl
from jax.experimental.pallas import tpu as pltpu
from jax.experimental.pallas import tpu_sc as plsc

B, S, D = 4, 4096, 2048
NTOK = B * S
NWORKERS = 32
TOK_PER_W = NTOK // NWORKERS
CHUNK = 16
NCHUNK = TOK_PER_W // CHUNK
NVEC = D // 16
EPS = 1e-5


_GDN = lax.GatherDimensionNumbers(
    offset_dims=(), collapsed_slice_dims=(0,), start_index_map=(0,))


def _lane_shuffle(v, idx):
    return lax.gather(v, idx[:, None], dimension_numbers=_GDN, slice_sizes=(1,),
                      mode=lax.GatherScatterMode.PROMISE_IN_BOUNDS)


def _lane_sum(v, lanes):
    for sh in (8, 4, 2, 1):
        v = v + _lane_shuffle(v, lanes ^ sh)
    return v


def _rsqrt_newton(v):
    bits = lax.bitcast_convert_type(v, jnp.int32)
    y = lax.bitcast_convert_type(jnp.int32(0x5F3759DF) - (bits >> 1), jnp.float32)
    for _ in range(3):
        y = y * (1.5 - 0.5 * v * y * y)
    return y


def _sc_body(h_hbm, idx_hbm, w0_hbm, b0_hbm, w1_hbm, b1_hbm, out_hbm,
             buf, w0_v, b0_v, w1_v, b1_v, idx_v, in_sem, out_sem):
    ncores = plsc.get_sparse_core_info().num_cores
    wid = lax.axis_index("s") * ncores + lax.axis_index("c")
    tok0 = wid * TOK_PER_W

    pltpu.sync_copy(w0_hbm, w0_v)
    pltpu.sync_copy(b0_hbm, b0_v)
    pltpu.sync_copy(w1_hbm, w1_v)
    pltpu.sync_copy(b1_hbm, b1_v)
    pltpu.sync_copy(idx_hbm.at[pl.ds(tok0, TOK_PER_W)], idx_v)

    lanes = lax.iota(jnp.int32, 16)
    zero16 = jnp.zeros((16,), jnp.float32)

    NSEG = 16
    JSEG = NVEC // NSEG
    CD = CHUNK * D

    def hbm_in(c):
        return h_hbm.at[pl.ds((tok0 + c * CHUNK) * D, CD)]

    def hbm_out(c):
        return out_hbm.at[pl.ds((tok0 + c * CHUNK) * D, CD)]

    def slot(c):
        return buf.at[pl.ds(lax.rem(c, 3) * CD, CD)]

    pltpu.async_copy(hbm_in(0), slot(0), in_sem)
    pltpu.async_copy(hbm_in(1), slot(1), in_sem)

    def chunk_body(c, _):
        boff = lax.rem(c, 3) * CD
        pltpu.make_async_copy(hbm_in(c), slot(c), in_sem).wait()

        def tok_body(t, carry):
            mu_v, var_v = carry
            row = boff + t * D
            acc = [zero16, zero16, zero16, zero16]
            for j in range(NVEC):
                x = buf[pl.ds(row + j * 16, 16)]
                k = j & 1
                acc[k] = acc[k] + x
                acc[2 + k] = acc[2 + k] + x * x
            mu = _lane_sum(acc[0] + acc[1], lanes) * (1.0 / D)
            var = _lane_sum(acc[2] + acc[3], lanes) * (1.0 / D) - mu * mu
            m = lanes == t
            return jnp.where(m, mu, mu_v), jnp.where(m, var, var_v)

        mu_v, var_v = lax.fori_loop(0, CHUNK, tok_body, (zero16, zero16))
        rstd_v = _rsqrt_newton(var_v + EPS)
        tv_v = idx_v[pl.ds(c * CHUNK, 16)]

        def seg_body(s, _):
            jb = s * (JSEG * 16)
            w0r = [w0_v[pl.ds(jb + k * 16, 16)] for k in range(JSEG)]
            w1r = [w1_v[pl.ds(jb + k * 16, 16)] for k in range(JSEG)]
            b0r = [b0_v[pl.ds(jb + k * 16, 16)] for k in range(JSEG)]
            b1r = [b1_v[pl.ds(jb + k * 16, 16)] for k in range(JSEG)]
            for t in range(CHUNK):
                tfull = jnp.full((16,), t, jnp.int32)
                mu_s = _lane_shuffle(mu_v, tfull)
                rstd_s = _lane_shuffle(rstd_v, tfull)
                sel1 = _lane_shuffle(tv_v, tfull) != 0.0
                off = boff + t * D + jb
                for k in range(JSEG):
                    x = buf[pl.ds(off + k * 16, 16)]
                    wj = jnp.where(sel1, w1r[k], w0r[k])
                    bj = jnp.where(sel1, b1r[k], b0r[k])
                    buf[pl.ds(off + k * 16, 16)] = (x - mu_s) * rstd_s * wj + bj
            return 0

        lax.fori_loop(0, NSEG, seg_body, 0)

        pltpu.async_copy(slot(c), hbm_out(c), out_sem)
        @pl.when(c >= 1)
        def _():
            pltpu.make_async_copy(slot(c - 1), hbm_out(c - 1), out_sem).wait()

        @pl.when(c + 2 < NCHUNK)
        def _():
            pltpu.async_copy(hbm_in(c + 2), slot(c + 2), in_sem)

        return 0

    lax.fori_loop(0, NCHUNK, chunk_body, 0)
    pltpu.make_async_copy(slot(NCHUNK - 1), hbm_out(NCHUNK - 1), out_sem).wait()


@jax.jit
def kernel(hidden_states, multiway_indices, ln0_w, ln0_b, ln1_w, ln1_b):
    h_flat = hidden_states.reshape(-1)
    idx_flat = multiway_indices.reshape(-1).astype(jnp.float32)

    mesh = plsc.VectorSubcoreMesh(core_axis_name="c", subcore_axis_name="s")
    run = pl.kernel(
        _sc_body,
        out_type=jax.ShapeDtypeStruct((NTOK * D,), jnp.float32),
        mesh=mesh,
        compiler_params=pltpu.CompilerParams(needs_layout_passes=False),
        scratch_types=[
            pltpu.VMEM((3 * CHUNK * D,), jnp.float32),
            pltpu.VMEM((D,), jnp.float32),
            pltpu.VMEM((D,), jnp.float32),
            pltpu.VMEM((D,), jnp.float32),
            pltpu.VMEM((D,), jnp.float32),
            pltpu.VMEM((TOK_PER_W,), jnp.float32),
            pltpu.SemaphoreType.DMA,
            pltpu.SemaphoreType.DMA,
        ],
    )
    out = run(h_flat, idx_flat, ln0_w, ln0_b, ln1_w, ln1_b)
    return out.reshape(B, S, D)
